# baseline (speedup 1.0000x reference)
import functools

import jax
import jax.numpy as jnp
from jax import lax
from jax.experimental import pallas as pl
from jax.experimental.pallas import tpu as pltpu
from jax.experimental.pallas import tpu_sc as plsc

_INFO = plsc.get_sparse_core_info()
_NC, _NS = _INFO.num_cores, _INFO.num_subcores
_NW = _NC * _NS


def _make_sc_copy(T, D, dtype):
    rows = T // _NW
    mesh = plsc.VectorSubcoreMesh(core_axis_name="c", subcore_axis_name="s")

    @functools.partial(
        pl.kernel,
        mesh=mesh,
        out_type=jax.ShapeDtypeStruct((T, D), dtype),
        scratch_types=[
            pltpu.VMEM((rows, D), dtype),
            pltpu.SemaphoreType.DMA,
        ],
    )
    def _sc_copy(pe_hbm, out_hbm, buf, sem):
        wid = lax.axis_index("s") * _NC + lax.axis_index("c")
        base = wid * rows
        pltpu.async_copy(pe_hbm.at[pl.ds(base, rows), :], buf, sem).wait()
        pltpu.sync_copy(buf, out_hbm.at[pl.ds(base, rows), :])

    return _sc_copy


def kernel(idx, pe):
    T = idx.shape[1]
    out = _make_sc_copy(T, pe.shape[1], pe.dtype)(pe)
    return out[None, :, :]
